# Initial kernel scaffold; baseline (speedup 1.0000x reference)
import jax
import jax.numpy as jnp
from jax.experimental import pallas as pl


def kernel(x, edge_index, W1, b1, W2, b2):
    raise NotImplementedError("write your pallas kernel here")


# speedup vs baseline: 10.0316x; 10.0316x over previous
import functools

import jax
import jax.numpy as jnp
from jax import lax
from jax.experimental import pallas as pl
from jax.experimental.pallas import tpu as pltpu
from jax.experimental.pallas import tpu_sc as plsc

N = 10000
D = 128
HD = D // 2
E = 320000
NC = 2
NS = 16
NW = NC * NS
EPW = E // NW
EPAD = 10240
NROW = EPAD // 128
NPAD = 10240
CHUNK = 128
RPS = NPAD // NS


def _deg_body(dst_hbm, ones_hbm, zeros_hbm, out_hbm, dst_v, ones_v, deg_sp):
    cid = lax.axis_index("c")
    sid = lax.axis_index("s")
    wid = sid * NC + cid
    pltpu.sync_copy(dst_hbm.at[wid], dst_v)
    pltpu.sync_copy(ones_hbm, ones_v)
    pltpu.sync_copy(zeros_hbm, deg_sp.at[pl.ds(sid * RPS, RPS)])
    plsc.subcore_barrier()

    @pl.loop(0, NROW)
    def _scatter_ones(j):
        pltpu.sync_copy(ones_v, deg_sp.at[dst_v.at[j]], add=True)

    plsc.subcore_barrier()
    pltpu.sync_copy(
        deg_sp.at[pl.ds(sid * RPS, RPS)],
        out_hbm.at[cid, pl.ds(sid * RPS, RPS)],
    )


def _prop_body(g_hbm, src_hbm, dst_hbm, zeros_hbm, out_hbm,
               src_v, dst_v, buf, acc_sp, sem0, sem1):
    cid = lax.axis_index("c")
    sid = lax.axis_index("s")
    wid = sid * NC + cid
    sems = (sem0, sem1)

    pltpu.sync_copy(src_hbm.at[wid], src_v)
    pltpu.sync_copy(dst_hbm.at[wid], dst_v)
    pltpu.sync_copy(zeros_hbm, acc_sp.at[pl.ds(sid * RPS, RPS)])
    plsc.subcore_barrier()

    for b in range(2):
        pltpu.async_copy(g_hbm.at[src_v.at[b]], buf.at[b], sems[b])

    @pl.loop(0, NROW - 2, step=2)
    def _edge_chunks(j):
        for b in range(2):
            c = j + b
            pltpu.make_async_copy(
                g_hbm.at[src_v.at[c]], buf.at[b], sems[b]).wait()
            pltpu.sync_copy(buf.at[b], acc_sp.at[dst_v.at[c]], add=True)
            pltpu.async_copy(g_hbm.at[src_v.at[c + 2]], buf.at[b], sems[b])

    for b in range(2):
        c = NROW - 2 + b
        pltpu.make_async_copy(g_hbm.at[src_v.at[c]], buf.at[b], sems[b]).wait()
        pltpu.sync_copy(buf.at[b], acc_sp.at[dst_v.at[c]], add=True)

    plsc.subcore_barrier()
    pltpu.sync_copy(
        acc_sp.at[pl.ds(sid * RPS, RPS)],
        out_hbm.at[cid, pl.ds(sid * RPS, RPS)],
    )


@functools.cache
def _sc_kernels():
    mesh = plsc.VectorSubcoreMesh(
        core_axis_name="c", subcore_axis_name="s",
        num_cores=NC, num_subcores=NS)
    params = pltpu.CompilerParams(use_tc_tiling_on_sc=False)
    deg_kernel = pl.kernel(
        _deg_body,
        out_type=jax.ShapeDtypeStruct((NC, NPAD, 16), jnp.float32),
        mesh=mesh,
        compiler_params=params,
        scratch_types=[
            pltpu.VMEM((NROW, 128), jnp.int32),
            pltpu.VMEM((128, 16), jnp.float32),
            pltpu.VMEM_SHARED((NPAD, 16), jnp.float32),
        ],
    )
    prop_kernel = pl.kernel(
        _prop_body,
        out_type=jax.ShapeDtypeStruct((NC, NPAD, HD), jnp.float32),
        mesh=mesh,
        compiler_params=params,
        scratch_types=[
            pltpu.VMEM((NROW, 128), jnp.int32),
            pltpu.VMEM((NROW, 128), jnp.int32),
            pltpu.VMEM((2, CHUNK, HD), jnp.float32),
            pltpu.VMEM_SHARED((NPAD, HD), jnp.float32),
            pltpu.SemaphoreType.DMA,
            pltpu.SemaphoreType.DMA,
        ],
    )
    return deg_kernel, prop_kernel


BR = 1000
GRID = N // BR


def _dis_block(degp):
    deg = degp[0, :, 0:1] + degp[1, :, 0:1] + 1.0
    return lax.rsqrt(deg)


def _merge(sp_lo, sp_hi):
    return jnp.concatenate([sp_lo[0] + sp_lo[1], sp_hi[0] + sp_hi[1]], axis=1)


def _mm1_body(x_ref, w_ref, degp_ref, glo_ref, ghi_ref):
    dis = _dis_block(degp_ref[...])
    g = dis * jnp.dot(x_ref[...], w_ref[...],
                      preferred_element_type=jnp.float32)
    glo_ref[...] = g[:, :HD]
    ghi_ref[...] = g[:, HD:]


def _mm2_body(slo_ref, shi_ref, glo_ref, ghi_ref, degp_ref, w_ref, b1_ref,
              g2lo_ref, g2hi_ref):
    dis = _dis_block(degp_ref[...])
    g1 = jnp.concatenate([glo_ref[...], ghi_ref[...]], axis=1)
    pre = dis * (_merge(slo_ref[...], shi_ref[...]) + g1) + b1_ref[...]
    x2 = jnp.maximum(pre, 0.0)
    g2 = dis * jnp.dot(x2, w_ref[...], preferred_element_type=jnp.float32)
    g2lo_ref[...] = g2[:, :HD]
    g2hi_ref[...] = g2[:, HD:]


def _fin_body(slo_ref, shi_ref, glo_ref, ghi_ref, degp_ref, b2_ref, out_ref):
    dis = _dis_block(degp_ref[...])
    g2 = jnp.concatenate([glo_ref[...], ghi_ref[...]], axis=1)
    out_ref[...] = dis * (_merge(slo_ref[...], shi_ref[...]) + g2) + b2_ref[...]


_half_spec = pl.BlockSpec((BR, HD), lambda i: (i, 0))
_degp_spec = pl.BlockSpec((NC, BR, 16), lambda i: (0, i, 0))
_part_spec = pl.BlockSpec((NC, BR, HD), lambda i: (0, i, 0))
_row_spec = pl.BlockSpec((BR, D), lambda i: (i, 0))
_w_spec = pl.BlockSpec((D, D), lambda i: (0, 0))
_b_spec = pl.BlockSpec((1, D), lambda i: (0, 0))
_half_sds = jax.ShapeDtypeStruct((N, HD), jnp.float32)

_mm1 = pl.pallas_call(
    _mm1_body,
    grid=(GRID,),
    in_specs=[_row_spec, _w_spec, _degp_spec],
    out_specs=[_half_spec, _half_spec],
    out_shape=[_half_sds, _half_sds],
)

_mm2 = pl.pallas_call(
    _mm2_body,
    grid=(GRID,),
    in_specs=[_part_spec, _part_spec, _half_spec, _half_spec, _degp_spec,
              _w_spec, _b_spec],
    out_specs=[_half_spec, _half_spec],
    out_shape=[_half_sds, _half_sds],
)

_fin = pl.pallas_call(
    _fin_body,
    grid=(GRID,),
    in_specs=[_part_spec, _part_spec, _half_spec, _half_spec, _degp_spec,
              _b_spec],
    out_specs=_row_spec,
    out_shape=jax.ShapeDtypeStruct((N, D), jnp.float32),
)


def kernel(x, edge_index, W1, b1, W2, b2):
    e = edge_index.astype(jnp.int32)
    pad_src = jnp.zeros((NW, EPAD - EPW), jnp.int32)
    pad_dst = jnp.full((NW, EPAD - EPW), N, jnp.int32)
    src = jnp.concatenate(
        [e[0].reshape(NW, EPW), pad_src], axis=1).reshape(NW, NROW, 128)
    dst = jnp.concatenate(
        [e[1].reshape(NW, EPW), pad_dst], axis=1).reshape(NW, NROW, 128)
    ones16 = jnp.ones((128, 16), jnp.float32)
    zeros16 = jnp.zeros((RPS, 16), jnp.float32)
    zerosH = jnp.zeros((RPS, HD), jnp.float32)

    deg_kernel, prop_kernel = _sc_kernels()
    degp = deg_kernel(dst, ones16, zeros16)
    g1lo, g1hi = _mm1(x, W1, degp)
    s1lo = prop_kernel(g1lo, src, dst, zerosH)
    s1hi = prop_kernel(g1hi, src, dst, zerosH)
    g2lo, g2hi = _mm2(s1lo, s1hi, g1lo, g1hi, degp, W2, b1.reshape(1, D))
    s2lo = prop_kernel(g2lo, src, dst, zerosH)
    s2hi = prop_kernel(g2hi, src, dst, zerosH)
    return _fin(s2lo, s2hi, g2lo, g2hi, degp, b2.reshape(1, D))


# speedup vs baseline: 10.1446x; 1.0113x over previous
import functools

import jax
import jax.numpy as jnp
from jax import lax
from jax.experimental import pallas as pl
from jax.experimental.pallas import tpu as pltpu
from jax.experimental.pallas import tpu_sc as plsc

N = 10000
D = 128
HD = D // 2
E = 320000
NC = 2
NS = 16
NW = NC * NS
EPW = E // NW
EPAD = 10240
NROW = EPAD // 128
NPAD = 10240
CHUNK = 128
RPS = NPAD // NS


def _deg_body(dst_hbm, ones_hbm, zeros_hbm, out_hbm, dst_v, ones_v, deg_sp):
    cid = lax.axis_index("c")
    sid = lax.axis_index("s")
    wid = sid * NC + cid
    pltpu.sync_copy(dst_hbm.at[wid], dst_v)
    pltpu.sync_copy(ones_hbm, ones_v)
    pltpu.sync_copy(zeros_hbm, deg_sp.at[pl.ds(sid * RPS, RPS)])
    plsc.subcore_barrier()

    @pl.loop(0, NROW)
    def _scatter_ones(j):
        pltpu.sync_copy(ones_v, deg_sp.at[dst_v.at[j]], add=True)

    plsc.subcore_barrier()
    pltpu.sync_copy(
        deg_sp.at[pl.ds(sid * RPS, RPS)],
        out_hbm.at[cid, pl.ds(sid * RPS, RPS)],
    )


def _prop_body(g_hbm, src_hbm, dst_hbm, zeros_hbm, out_hbm,
               src_v, dst_v, buf, acc_sp,
               gsem0, gsem1, gsem2, gsem3, ssem0, ssem1, ssem2, ssem3):
    cid = lax.axis_index("c")
    sid = lax.axis_index("s")
    wid = sid * NC + cid
    gsems = (gsem0, gsem1, gsem2, gsem3)
    ssems = (ssem0, ssem1, ssem2, ssem3)

    pltpu.sync_copy(src_hbm.at[wid], src_v)
    pltpu.sync_copy(dst_hbm.at[wid], dst_v)
    pltpu.sync_copy(zeros_hbm, acc_sp.at[pl.ds(sid * RPS, RPS)])
    plsc.subcore_barrier()

    for b in range(2):
        pltpu.async_copy(g_hbm.at[src_v.at[b]], buf.at[b], gsems[b])

    @pl.loop(0, NROW // 4)
    def _edge_chunks(j):
        for b in range(4):
            c = 4 * j + b
            b2 = (b + 2) % 4
            pltpu.make_async_copy(
                g_hbm.at[src_v.at[c]], buf.at[b], gsems[b]).wait()
            pltpu.async_copy(
                buf.at[b], acc_sp.at[dst_v.at[c]], ssems[b], add=True)

            @pl.when(c >= 2)
            def _():
                pltpu.make_async_copy(
                    buf.at[b2], acc_sp.at[dst_v.at[c]], ssems[b2]).wait()

            @pl.when(c < NROW - 2)
            def _():
                pltpu.async_copy(
                    g_hbm.at[src_v.at[c + 2]], buf.at[b2], gsems[b2])

    for b in (2, 3):
        pltpu.make_async_copy(
            buf.at[b], acc_sp.at[dst_v.at[0]], ssems[b]).wait()

    plsc.subcore_barrier()
    pltpu.sync_copy(
        acc_sp.at[pl.ds(sid * RPS, RPS)],
        out_hbm.at[cid, pl.ds(sid * RPS, RPS)],
    )


@functools.cache
def _sc_kernels():
    mesh = plsc.VectorSubcoreMesh(
        core_axis_name="c", subcore_axis_name="s",
        num_cores=NC, num_subcores=NS)
    params = pltpu.CompilerParams(use_tc_tiling_on_sc=False)
    deg_kernel = pl.kernel(
        _deg_body,
        out_type=jax.ShapeDtypeStruct((NC, NPAD, 16), jnp.float32),
        mesh=mesh,
        compiler_params=params,
        scratch_types=[
            pltpu.VMEM((NROW, 128), jnp.int32),
            pltpu.VMEM((128, 16), jnp.float32),
            pltpu.VMEM_SHARED((NPAD, 16), jnp.float32),
        ],
    )
    prop_kernel = pl.kernel(
        _prop_body,
        out_type=jax.ShapeDtypeStruct((NC, NPAD, HD), jnp.float32),
        mesh=mesh,
        compiler_params=params,
        scratch_types=[
            pltpu.VMEM((NROW, 128), jnp.int32),
            pltpu.VMEM((NROW, 128), jnp.int32),
            pltpu.VMEM((4, CHUNK, HD), jnp.float32),
            pltpu.VMEM_SHARED((NPAD, HD), jnp.float32),
        ] + [pltpu.SemaphoreType.DMA] * 8,
    )
    return deg_kernel, prop_kernel


BR = 1000
GRID = N // BR


def _dis_block(degp):
    deg = degp[0, :, 0:1] + degp[1, :, 0:1] + 1.0
    return lax.rsqrt(deg)


def _merge(sp_lo, sp_hi):
    return jnp.concatenate([sp_lo[0] + sp_lo[1], sp_hi[0] + sp_hi[1]], axis=1)


def _mm1_body(x_ref, w_ref, degp_ref, glo_ref, ghi_ref):
    dis = _dis_block(degp_ref[...])
    g = dis * jnp.dot(x_ref[...], w_ref[...],
                      preferred_element_type=jnp.float32)
    glo_ref[...] = g[:, :HD]
    ghi_ref[...] = g[:, HD:]


def _mm2_body(slo_ref, shi_ref, glo_ref, ghi_ref, degp_ref, w_ref, b1_ref,
              g2lo_ref, g2hi_ref):
    dis = _dis_block(degp_ref[...])
    g1 = jnp.concatenate([glo_ref[...], ghi_ref[...]], axis=1)
    pre = dis * (_merge(slo_ref[...], shi_ref[...]) + g1) + b1_ref[...]
    x2 = jnp.maximum(pre, 0.0)
    g2 = dis * jnp.dot(x2, w_ref[...], preferred_element_type=jnp.float32)
    g2lo_ref[...] = g2[:, :HD]
    g2hi_ref[...] = g2[:, HD:]


def _fin_body(slo_ref, shi_ref, glo_ref, ghi_ref, degp_ref, b2_ref, out_ref):
    dis = _dis_block(degp_ref[...])
    g2 = jnp.concatenate([glo_ref[...], ghi_ref[...]], axis=1)
    out_ref[...] = dis * (_merge(slo_ref[...], shi_ref[...]) + g2) + b2_ref[...]


_half_spec = pl.BlockSpec((BR, HD), lambda i: (i, 0))
_degp_spec = pl.BlockSpec((NC, BR, 16), lambda i: (0, i, 0))
_part_spec = pl.BlockSpec((NC, BR, HD), lambda i: (0, i, 0))
_row_spec = pl.BlockSpec((BR, D), lambda i: (i, 0))
_w_spec = pl.BlockSpec((D, D), lambda i: (0, 0))
_b_spec = pl.BlockSpec((1, D), lambda i: (0, 0))
_half_sds = jax.ShapeDtypeStruct((N, HD), jnp.float32)

_mm1 = pl.pallas_call(
    _mm1_body,
    grid=(GRID,),
    in_specs=[_row_spec, _w_spec, _degp_spec],
    out_specs=[_half_spec, _half_spec],
    out_shape=[_half_sds, _half_sds],
)

_mm2 = pl.pallas_call(
    _mm2_body,
    grid=(GRID,),
    in_specs=[_part_spec, _part_spec, _half_spec, _half_spec, _degp_spec,
              _w_spec, _b_spec],
    out_specs=[_half_spec, _half_spec],
    out_shape=[_half_sds, _half_sds],
)

_fin = pl.pallas_call(
    _fin_body,
    grid=(GRID,),
    in_specs=[_part_spec, _part_spec, _half_spec, _half_spec, _degp_spec,
              _b_spec],
    out_specs=_row_spec,
    out_shape=jax.ShapeDtypeStruct((N, D), jnp.float32),
)


def kernel(x, edge_index, W1, b1, W2, b2):
    e = edge_index.astype(jnp.int32)
    pad_src = jnp.zeros((NW, EPAD - EPW), jnp.int32)
    pad_dst = jnp.full((NW, EPAD - EPW), N, jnp.int32)
    src = jnp.concatenate(
        [e[0].reshape(NW, EPW), pad_src], axis=1).reshape(NW, NROW, 128)
    dst = jnp.concatenate(
        [e[1].reshape(NW, EPW), pad_dst], axis=1).reshape(NW, NROW, 128)
    ones16 = jnp.ones((128, 16), jnp.float32)
    zeros16 = jnp.zeros((RPS, 16), jnp.float32)
    zerosH = jnp.zeros((RPS, HD), jnp.float32)

    deg_kernel, prop_kernel = _sc_kernels()
    degp = deg_kernel(dst, ones16, zeros16)
    g1lo, g1hi = _mm1(x, W1, degp)
    s1lo = prop_kernel(g1lo, src, dst, zerosH)
    s1hi = prop_kernel(g1hi, src, dst, zerosH)
    g2lo, g2hi = _mm2(s1lo, s1hi, g1lo, g1hi, degp, W2, b1.reshape(1, D))
    s2lo = prop_kernel(g2lo, src, dst, zerosH)
    s2hi = prop_kernel(g2hi, src, dst, zerosH)
    return _fin(s2lo, s2hi, g2lo, g2hi, degp, b2.reshape(1, D))


# speedup vs baseline: 14.9623x; 1.4749x over previous
import functools

import jax
import jax.numpy as jnp
from jax import lax
from jax.experimental import pallas as pl
from jax.experimental.pallas import tpu as pltpu
from jax.experimental.pallas import tpu_sc as plsc

N = 10000
D = 128
HD = D // 2
E = 320000
NC = 2
NS = 16
NW = NC * NS
EPW = E // NW
EPAD = 10240
NROW = EPAD // 128
EPS = E // NS
NROW2 = 2 * NROW
NPAD = 10240
CHUNK = 128
RPS = NPAD // NS


def _deg_body(dst_hbm, ones_hbm, zeros_hbm, out_hbm, dst_v, ones_v, deg_sp):
    cid = lax.axis_index("c")
    sid = lax.axis_index("s")
    wid = sid * NC + cid
    pltpu.sync_copy(dst_hbm.at[wid], dst_v)
    pltpu.sync_copy(ones_hbm, ones_v)
    pltpu.sync_copy(zeros_hbm, deg_sp.at[pl.ds(sid * RPS, RPS)])
    plsc.subcore_barrier()

    @pl.loop(0, NROW)
    def _scatter_ones(j):
        pltpu.sync_copy(ones_v, deg_sp.at[dst_v.at[j]], add=True)

    plsc.subcore_barrier()
    pltpu.sync_copy(
        deg_sp.at[pl.ds(sid * RPS, RPS)],
        out_hbm.at[cid, pl.ds(sid * RPS, RPS)],
    )


def _prop_body(g_hbm, src_hbm, dst_hbm, zeros_hbm, out_hbm,
               src_v, dst_v, buf, acc_sp,
               gsem0, gsem1, gsem2, gsem3, ssem0, ssem1, ssem2, ssem3):
    cid = lax.axis_index("c")
    sid = lax.axis_index("s")
    gsems = (gsem0, gsem1, gsem2, gsem3)
    ssems = (ssem0, ssem1, ssem2, ssem3)

    pltpu.sync_copy(src_hbm.at[cid, sid], src_v)
    pltpu.sync_copy(dst_hbm.at[sid], dst_v)
    pltpu.sync_copy(zeros_hbm, acc_sp.at[pl.ds(sid * RPS, RPS)])
    plsc.subcore_barrier()

    for b in range(2):
        pltpu.async_copy(g_hbm.at[src_v.at[b]], buf.at[b], gsems[b])

    @pl.loop(0, NROW2 // 4)
    def _edge_chunks(j):
        for b in range(4):
            c = 4 * j + b
            b2 = (b + 2) % 4
            pltpu.make_async_copy(
                g_hbm.at[src_v.at[c]], buf.at[b], gsems[b]).wait()
            pltpu.async_copy(
                buf.at[b], acc_sp.at[dst_v.at[c]], ssems[b], add=True)

            @pl.when(c >= 2)
            def _():
                pltpu.make_async_copy(
                    buf.at[b2], acc_sp.at[dst_v.at[c]], ssems[b2]).wait()

            @pl.when(c < NROW2 - 2)
            def _():
                pltpu.async_copy(
                    g_hbm.at[src_v.at[c + 2]], buf.at[b2], gsems[b2])

    for b in (2, 3):
        pltpu.make_async_copy(
            buf.at[b], acc_sp.at[dst_v.at[0]], ssems[b]).wait()

    plsc.subcore_barrier()
    pltpu.sync_copy(
        acc_sp.at[pl.ds(sid * RPS, RPS)],
        out_hbm.at[cid, pl.ds(sid * RPS, RPS)],
    )


@functools.cache
def _sc_kernels():
    mesh = plsc.VectorSubcoreMesh(
        core_axis_name="c", subcore_axis_name="s",
        num_cores=NC, num_subcores=NS)
    params = pltpu.CompilerParams(use_tc_tiling_on_sc=False)
    deg_kernel = pl.kernel(
        _deg_body,
        out_type=jax.ShapeDtypeStruct((NC, NPAD, 16), jnp.float32),
        mesh=mesh,
        compiler_params=params,
        scratch_types=[
            pltpu.VMEM((NROW, 128), jnp.int32),
            pltpu.VMEM((128, 16), jnp.float32),
            pltpu.VMEM_SHARED((NPAD, 16), jnp.float32),
        ],
    )
    prop_kernel = pl.kernel(
        _prop_body,
        out_type=jax.ShapeDtypeStruct((NC, NPAD, HD), jnp.float32),
        mesh=mesh,
        compiler_params=params,
        scratch_types=[
            pltpu.VMEM((NROW2, 128), jnp.int32),
            pltpu.VMEM((NROW2, 128), jnp.int32),
            pltpu.VMEM((4, CHUNK, HD), jnp.float32),
            pltpu.VMEM_SHARED((NPAD, HD), jnp.float32),
        ] + [pltpu.SemaphoreType.DMA] * 8,
    )
    return deg_kernel, prop_kernel


BR = 1000
GRID = N // BR


def _dis_block(degp):
    deg = degp[0, :, 0:1] + degp[1, :, 0:1] + 1.0
    return lax.rsqrt(deg)


def _cat(h):
    return jnp.concatenate([h[0], h[1]], axis=1)


def _mm1_body(x_ref, w_ref, degp_ref, g_ref):
    dis = _dis_block(degp_ref[...])
    g = dis * jnp.dot(x_ref[...], w_ref[...],
                      preferred_element_type=jnp.float32)
    g_ref[0] = g[:, :HD]
    g_ref[1] = g[:, HD:]


def _mm2_body(sp_ref, g1_ref, degp_ref, w_ref, b1_ref, g2_ref):
    dis = _dis_block(degp_ref[...])
    pre = dis * (_cat(sp_ref[...]) + _cat(g1_ref[...])) + b1_ref[...]
    x2 = jnp.maximum(pre, 0.0)
    g2 = dis * jnp.dot(x2, w_ref[...], preferred_element_type=jnp.float32)
    g2_ref[0] = g2[:, :HD]
    g2_ref[1] = g2[:, HD:]


def _fin_body(sp_ref, g2_ref, degp_ref, b2_ref, out_ref):
    dis = _dis_block(degp_ref[...])
    out_ref[...] = dis * (_cat(sp_ref[...]) + _cat(g2_ref[...])) + b2_ref[...]


_degp_spec = pl.BlockSpec((NC, BR, 16), lambda i: (0, i, 0))
_gh_spec = pl.BlockSpec((2, BR, HD), lambda i: (0, i, 0))
_row_spec = pl.BlockSpec((BR, D), lambda i: (i, 0))
_w_spec = pl.BlockSpec((D, D), lambda i: (0, 0))
_b_spec = pl.BlockSpec((1, D), lambda i: (0, 0))
_gh_sds = jax.ShapeDtypeStruct((2, N, HD), jnp.float32)

_mm1 = pl.pallas_call(
    _mm1_body,
    grid=(GRID,),
    in_specs=[_row_spec, _w_spec, _degp_spec],
    out_specs=_gh_spec,
    out_shape=_gh_sds,
)

_mm2 = pl.pallas_call(
    _mm2_body,
    grid=(GRID,),
    in_specs=[_gh_spec, _gh_spec, _degp_spec, _w_spec, _b_spec],
    out_specs=_gh_spec,
    out_shape=_gh_sds,
)

_fin = pl.pallas_call(
    _fin_body,
    grid=(GRID,),
    in_specs=[_gh_spec, _gh_spec, _degp_spec, _b_spec],
    out_specs=_row_spec,
    out_shape=jax.ShapeDtypeStruct((N, D), jnp.float32),
)


def kernel(x, edge_index, W1, b1, W2, b2):
    e = edge_index.astype(jnp.int32)
    pad_src = jnp.zeros((NS, NROW2 * 128 - EPS), jnp.int32)
    pad_dst = jnp.full((NS, NROW2 * 128 - EPS), N, jnp.int32)
    src1 = jnp.concatenate(
        [e[0].reshape(NS, EPS), pad_src], axis=1).reshape(NS, NROW2, 128)
    srcs = jnp.stack([src1, src1 + N])
    dst = jnp.concatenate(
        [e[1].reshape(NS, EPS), pad_dst], axis=1).reshape(NS, NROW2, 128)
    dstw = dst.reshape(NW, NROW, 128)
    ones16 = jnp.ones((128, 16), jnp.float32)
    zeros16 = jnp.zeros((RPS, 16), jnp.float32)
    zerosH = jnp.zeros((RPS, HD), jnp.float32)

    deg_kernel, prop_kernel = _sc_kernels()
    degp = deg_kernel(dstw, ones16, zeros16)
    g1 = _mm1(x, W1, degp)
    s1 = prop_kernel(g1.reshape(2 * N, HD), srcs, dst, zerosH)
    g2 = _mm2(s1, g1, degp, W2, b1.reshape(1, D))
    s2 = prop_kernel(g2.reshape(2 * N, HD), srcs, dst, zerosH)
    return _fin(s2, g2, degp, b2.reshape(1, D))
